# baseline (speedup 1.0000x reference)
import functools

import jax
import jax.numpy as jnp
from jax import lax
from jax.experimental import pallas as pl
from jax.experimental.pallas import tpu as pltpu
from jax.experimental.pallas import tpu_sc as plsc

N = 10000
D = 128
E = 320000
DT = 1.0
ALPHA = 1.0
GAMMA = 1.0

NC = 2
NS = 16
NW = NC * NS
CHUNK = 128
W = 8
NCHUNK = -(-E // (NW * CHUNK * 2 * W)) * 2 * W
EPT = NCHUNK * CHUNK
E_PAD = EPT * NW
NWIN = NCHUNK // W
RPT = 632
N_PAD = RPT * NS


def _sc_agg_body(src_hbm, dst_hbm, x_hbm, zero_hbm, out_hbm,
                 src_wa, dst_wa, src_wb, dst_wb, rows0_v, rows1_v, acc_sh,
                 gsem0, gsem1, ssem0, ssem1, isema, isemb):
    c = lax.axis_index("c")
    s = lax.axis_index("s")
    wid = s * NC + c
    row0 = pl.multiple_of(s * RPT, 8)
    pltpu.sync_copy(zero_hbm, acc_sh.at[pl.ds(row0, RPT)])

    def fetch_win(w, sv, dv, sem):
        pltpu.async_copy(src_hbm.at[wid * NWIN + w], sv, sem)
        pltpu.async_copy(dst_hbm.at[wid * NWIN + w], dv, sem)

    def wait_win(sv, dv, sem):
        pltpu.make_async_copy(src_hbm.at[0], sv, sem).wait()
        pltpu.make_async_copy(dst_hbm.at[0], dv, sem).wait()

    fetch_win(0, src_wa, dst_wa, isema)
    plsc.subcore_barrier()

    def process(sv, dv):
        for k in range(W):
            pltpu.async_copy(x_hbm.at[sv.at[k]], rows0_v, gsem0).wait()
            pltpu.sync_copy(rows0_v, acc_sh.at[dv.at[k]], add=True)

    def body(j, carry):
        wait_win(src_wa, dst_wa, isema)
        fetch_win(2 * j + 1, src_wb, dst_wb, isemb)
        process(src_wa, dst_wa)
        wait_win(src_wb, dst_wb, isemb)

        @pl.when(j < NWIN // 2 - 1)
        def _():
            fetch_win(2 * j + 2, src_wa, dst_wa, isema)

        process(src_wb, dst_wb)
        return carry

    lax.fori_loop(0, NWIN // 2, body, 0)
    plsc.subcore_barrier()
    pltpu.sync_copy(acc_sh.at[pl.ds(row0, RPT)],
                    out_hbm.at[c, pl.ds(row0, RPT)])


_sc_agg = functools.partial(
    pl.kernel,
    out_type=jax.ShapeDtypeStruct((NC, N_PAD, D), jnp.float32),
    mesh=plsc.VectorSubcoreMesh(core_axis_name="c", subcore_axis_name="s"),
    scratch_types=[
        pltpu.VMEM((W, CHUNK), jnp.int32),
        pltpu.VMEM((W, CHUNK), jnp.int32),
        pltpu.VMEM((W, CHUNK), jnp.int32),
        pltpu.VMEM((W, CHUNK), jnp.int32),
        pltpu.VMEM((CHUNK, D), jnp.float32),
        pltpu.VMEM((CHUNK, D), jnp.float32),
        pltpu.VMEM_SHARED((N_PAD, D), jnp.float32),
        pltpu.SemaphoreType.DMA,
        pltpu.SemaphoreType.DMA,
        pltpu.SemaphoreType.DMA,
        pltpu.SemaphoreType.DMA,
        pltpu.SemaphoreType.DMA,
        pltpu.SemaphoreType.DMA,
    ],
)(_sc_agg_body)


BLK = 1000


def _tc_update_body(p_ref, x_ref, y_ref, w_ref, b_ref, xo_ref, yo_ref):
    agg = p_ref[0] + p_ref[1]
    g = jnp.dot(agg, w_ref[...], preferred_element_type=jnp.float32) + b_ref[...]
    r = jnp.maximum(g, 0.0)
    x = x_ref[...]
    y = y_ref[...]
    ynew = y + DT * (r - ALPHA * y - GAMMA * x)
    xo_ref[...] = x + DT * ynew
    yo_ref[...] = ynew


def _tc_update(p, x, y, w, b):
    return pl.pallas_call(
        _tc_update_body,
        grid=(N // BLK,),
        in_specs=[
            pl.BlockSpec((NC, BLK, D), lambda i: (0, i, 0)),
            pl.BlockSpec((BLK, D), lambda i: (i, 0)),
            pl.BlockSpec((BLK, D), lambda i: (i, 0)),
            pl.BlockSpec((D, D), lambda i: (0, 0)),
            pl.BlockSpec((1, D), lambda i: (0, 0)),
        ],
        out_specs=[pl.BlockSpec((BLK, D), lambda i: (i, 0)),
                   pl.BlockSpec((BLK, D), lambda i: (i, 0))],
        out_shape=[jax.ShapeDtypeStruct((N, D), jnp.float32),
                   jax.ShapeDtypeStruct((N, D), jnp.float32)],
    )(p, x, y, w, b.reshape(1, D))


def kernel(X0, Y0, edge_index, W1, b1, W2, b2):
    src = edge_index[0].astype(jnp.int32)
    dst = edge_index[1].astype(jnp.int32)
    pad = E_PAD - E
    src = jnp.concatenate([src, jnp.zeros((pad,), jnp.int32)])
    dst = jnp.concatenate([dst, jnp.full((pad,), N, jnp.int32)])
    src = src.reshape(NW * NWIN, W, CHUNK)
    dst = dst.reshape(NW * NWIN, W, CHUNK)
    zero = jnp.zeros((RPT, D), jnp.float32)

    p1 = _sc_agg(src, dst, X0, zero)
    X1, Y1 = _tc_update(p1, X0, Y0, W1, b1)

    p2 = _sc_agg(src, dst, X1, zero)
    X2, Y2 = _tc_update(p2, X1, Y1, W2, b2)
    return (X2, Y2)


# speedup vs baseline: 1.1358x; 1.1358x over previous
import functools

import jax
import jax.numpy as jnp
from jax import lax
from jax.experimental import pallas as pl
from jax.experimental.pallas import tpu as pltpu
from jax.experimental.pallas import tpu_sc as plsc

N = 10000
D = 128
E = 320000
DT = 1.0
ALPHA = 1.0
GAMMA = 1.0

NC = 2
NS = 16
NW = NC * NS
CHUNK = 128
NCHUNK = -(-E // (NW * CHUNK * 2)) * 2
EPT = NCHUNK * CHUNK
E_PAD = EPT * NW
VPC = CHUNK // 16
SHIFT = 14
MASK = (1 << SHIFT) - 1
RPT = 632
N_PAD = RPT * NS


def _sc_agg_body(packed_hbm, x_hbm, zero_hbm, out_hbm,
                 packed_v, s0, d0, s1, d1, rows0_v, rows1_v, acc_sh,
                 gsem0, gsem1, ssem0, ssem1):
    c = lax.axis_index("c")
    s = lax.axis_index("s")
    wid = s * NC + c
    row0 = pl.multiple_of(s * RPT, 8)
    pltpu.sync_copy(zero_hbm, acc_sh.at[pl.ds(row0, RPT)])
    pltpu.sync_copy(packed_hbm.at[wid], packed_v)

    def unpack(i, sv, dv):
        for t in range(VPC):
            p = packed_v[i, pl.ds(t * 16, 16)]
            sv[pl.ds(t * 16, 16)] = jax.lax.shift_right_logical(p, SHIFT)
            dv[pl.ds(t * 16, 16)] = jax.lax.bitwise_and(p, MASK)

    plsc.subcore_barrier()

    unpack(0, s0, d0)
    pltpu.async_copy(x_hbm.at[s0], rows0_v, gsem0)
    unpack(1, s1, d1)
    pltpu.async_copy(x_hbm.at[s1], rows1_v, gsem1)

    def body(j, carry):
        i = j * 2
        pltpu.make_async_copy(x_hbm.at[s0], rows0_v, gsem0).wait()
        pltpu.async_copy(rows0_v, acc_sh.at[d0], ssem0, add=True)
        pltpu.make_async_copy(x_hbm.at[s1], rows1_v, gsem1).wait()
        pltpu.async_copy(rows1_v, acc_sh.at[d1], ssem1, add=True)
        pltpu.make_async_copy(rows0_v, acc_sh.at[d0], ssem0).wait()

        @pl.when(j < NCHUNK // 2 - 1)
        def _():
            unpack(i + 2, s0, d0)
            pltpu.async_copy(x_hbm.at[s0], rows0_v, gsem0)

        pltpu.make_async_copy(rows1_v, acc_sh.at[d1], ssem1).wait()

        @pl.when(j < NCHUNK // 2 - 1)
        def _():
            unpack(i + 3, s1, d1)
            pltpu.async_copy(x_hbm.at[s1], rows1_v, gsem1)

        return carry

    lax.fori_loop(0, NCHUNK // 2, body, 0)
    plsc.subcore_barrier()
    pltpu.sync_copy(acc_sh.at[pl.ds(row0, RPT)],
                    out_hbm.at[c, pl.ds(row0, RPT)])


_sc_agg = functools.partial(
    pl.kernel,
    out_type=jax.ShapeDtypeStruct((NC, N_PAD, D), jnp.float32),
    mesh=plsc.VectorSubcoreMesh(core_axis_name="c", subcore_axis_name="s"),
    scratch_types=[
        pltpu.VMEM((NCHUNK, CHUNK), jnp.int32),
        pltpu.VMEM((CHUNK,), jnp.int32),
        pltpu.VMEM((CHUNK,), jnp.int32),
        pltpu.VMEM((CHUNK,), jnp.int32),
        pltpu.VMEM((CHUNK,), jnp.int32),
        pltpu.VMEM((CHUNK, D), jnp.float32),
        pltpu.VMEM((CHUNK, D), jnp.float32),
        pltpu.VMEM_SHARED((N_PAD, D), jnp.float32),
        pltpu.SemaphoreType.DMA,
        pltpu.SemaphoreType.DMA,
        pltpu.SemaphoreType.DMA,
        pltpu.SemaphoreType.DMA,
    ],
)(_sc_agg_body)


BLK = 1000


def _tc_update_body(p_ref, x_ref, y_ref, w_ref, b_ref, xo_ref, yo_ref):
    agg = p_ref[0] + p_ref[1]
    g = jnp.dot(agg, w_ref[...], preferred_element_type=jnp.float32) + b_ref[...]
    r = jnp.maximum(g, 0.0)
    x = x_ref[...]
    y = y_ref[...]
    ynew = y + DT * (r - ALPHA * y - GAMMA * x)
    xo_ref[...] = x + DT * ynew
    yo_ref[...] = ynew


def _tc_update(p, x, y, w, b):
    return pl.pallas_call(
        _tc_update_body,
        grid=(N // BLK,),
        in_specs=[
            pl.BlockSpec((NC, BLK, D), lambda i: (0, i, 0)),
            pl.BlockSpec((BLK, D), lambda i: (i, 0)),
            pl.BlockSpec((BLK, D), lambda i: (i, 0)),
            pl.BlockSpec((D, D), lambda i: (0, 0)),
            pl.BlockSpec((1, D), lambda i: (0, 0)),
        ],
        out_specs=[pl.BlockSpec((BLK, D), lambda i: (i, 0)),
                   pl.BlockSpec((BLK, D), lambda i: (i, 0))],
        out_shape=[jax.ShapeDtypeStruct((N, D), jnp.float32),
                   jax.ShapeDtypeStruct((N, D), jnp.float32)],
    )(p, x, y, w, b.reshape(1, D))


def kernel(X0, Y0, edge_index, W1, b1, W2, b2):
    src = edge_index[0].astype(jnp.int32)
    dst = edge_index[1].astype(jnp.int32)
    pad = E_PAD - E
    src = jnp.concatenate([src, jnp.zeros((pad,), jnp.int32)])
    dst = jnp.concatenate([dst, jnp.full((pad,), N, jnp.int32)])
    packed = jnp.bitwise_or(jnp.left_shift(src, SHIFT), dst)
    packed = packed.reshape(NW, NCHUNK, CHUNK)
    zero = jnp.zeros((RPT, D), jnp.float32)

    p1 = _sc_agg(packed, X0, zero)
    X1, Y1 = _tc_update(p1, X0, Y0, W1, b1)

    p2 = _sc_agg(packed, X1, zero)
    X2, Y2 = _tc_update(p2, X1, Y1, W2, b2)
    return (X2, Y2)


# speedup vs baseline: 1.3770x; 1.2123x over previous
import functools

import jax
import jax.numpy as jnp
from jax import lax
from jax.experimental import pallas as pl
from jax.experimental.pallas import tpu as pltpu
from jax.experimental.pallas import tpu_sc as plsc

N = 10000
D = 128
E = 320000
DT = 1.0
ALPHA = 1.0
GAMMA = 1.0

NC = 2
NS = 16
NW = NC * NS
CHUNK = 128
EPT = -(-E // (NW * CHUNK)) * CHUNK
E_PAD = EPT * NW
NCHUNK = EPT // CHUNK
RPT = 632
N_PAD = RPT * NS


def _sc_agg_body(src_hbm, dst_hbm, x_hbm, zero_hbm, out_hbm,
                 src_v, dst_v, rows_v, acc_sh, sem):
    c = lax.axis_index("c")
    s = lax.axis_index("s")
    wid = s * NC + c
    row0 = pl.multiple_of(s * RPT, 8)
    pltpu.sync_copy(zero_hbm, acc_sh.at[pl.ds(row0, RPT)])
    pltpu.sync_copy(src_hbm.at[wid], src_v)
    pltpu.sync_copy(dst_hbm.at[wid], dst_v)
    plsc.subcore_barrier()

    def body(i, carry):
        pltpu.async_copy(x_hbm.at[src_v.at[i]], rows_v, sem).wait()
        pltpu.sync_copy(rows_v, acc_sh.at[dst_v.at[i]], add=True)
        return carry

    lax.fori_loop(0, NCHUNK, body, 0)
    plsc.subcore_barrier()
    pltpu.sync_copy(acc_sh.at[pl.ds(row0, RPT)],
                    out_hbm.at[c, pl.ds(row0, RPT)])


_sc_agg = functools.partial(
    pl.kernel,
    out_type=jax.ShapeDtypeStruct((NC, N_PAD, D), jnp.float32),
    mesh=plsc.VectorSubcoreMesh(core_axis_name="c", subcore_axis_name="s"),
    scratch_types=[
        pltpu.VMEM((NCHUNK, CHUNK), jnp.int32),
        pltpu.VMEM((NCHUNK, CHUNK), jnp.int32),
        pltpu.VMEM((CHUNK, D), jnp.float32),
        pltpu.VMEM_SHARED((N_PAD, D), jnp.float32),
        pltpu.SemaphoreType.DMA,
    ],
)(_sc_agg_body)


BLK = 1000


def _tc_update_body(p_ref, x_ref, y_ref, w_ref, b_ref, xo_ref, yo_ref):
    agg = p_ref[0] + p_ref[1]
    g = jnp.dot(agg, w_ref[...], preferred_element_type=jnp.float32) + b_ref[...]
    r = jnp.maximum(g, 0.0)
    x = x_ref[...]
    y = y_ref[...]
    ynew = y + DT * (r - ALPHA * y - GAMMA * x)
    xo_ref[...] = x + DT * ynew
    yo_ref[...] = ynew


def _tc_update(p, x, y, w, b):
    return pl.pallas_call(
        _tc_update_body,
        grid=(N // BLK,),
        in_specs=[
            pl.BlockSpec((NC, BLK, D), lambda i: (0, i, 0)),
            pl.BlockSpec((BLK, D), lambda i: (i, 0)),
            pl.BlockSpec((BLK, D), lambda i: (i, 0)),
            pl.BlockSpec((D, D), lambda i: (0, 0)),
            pl.BlockSpec((1, D), lambda i: (0, 0)),
        ],
        out_specs=[pl.BlockSpec((BLK, D), lambda i: (i, 0)),
                   pl.BlockSpec((BLK, D), lambda i: (i, 0))],
        out_shape=[jax.ShapeDtypeStruct((N, D), jnp.float32),
                   jax.ShapeDtypeStruct((N, D), jnp.float32)],
    )(p, x, y, w, b.reshape(1, D))


def kernel(X0, Y0, edge_index, W1, b1, W2, b2):
    src = edge_index[0].astype(jnp.int32)
    dst = edge_index[1].astype(jnp.int32)
    pad = E_PAD - E
    src = jnp.concatenate([src, jnp.zeros((pad,), jnp.int32)])
    dst = jnp.concatenate([dst, jnp.full((pad,), N, jnp.int32)])
    src = src.reshape(NW, NCHUNK, CHUNK)
    dst = dst.reshape(NW, NCHUNK, CHUNK)
    zero = jnp.zeros((RPT, D), jnp.float32)

    p1 = _sc_agg(src, dst, X0, zero)
    X1, Y1 = _tc_update(p1, X0, Y0, W1, b1)

    p2 = _sc_agg(src, dst, X1, zero)
    X2, Y2 = _tc_update(p2, X1, Y1, W2, b2)
    return (X2, Y2)
